# Initial kernel scaffold; baseline (speedup 1.0000x reference)
import jax
import jax.numpy as jnp
from jax.experimental import pallas as pl


def kernel(positives, negatives, r_head_base_points, r_head_widths, r_head_size_scales, r_tail_base_points, r_tail_widths, r_tail_size_scales, entity_bases, entity_bumps):
    raise NotImplementedError("write your pallas kernel here")


# speedup vs baseline: 6.2453x; 6.2453x over previous
import functools

import jax
import jax.numpy as jnp
from jax import lax
from jax.experimental import pallas as pl
from jax.experimental.pallas import tpu as pltpu
from jax.experimental.pallas import tpu_sc as plsc

_EMB = 128
_NB_REL = 100
_NB_ENT = 1000
_BATCH = 4096
_NB_NEG = 64

_NC = 2
_NS = 16
_NW = _NC * _NS
_C = 64


def _tables_body(rhb, rhw, rhs, rtb, rtw, rts, eb, ebp,
                 box_ref, t1_ref, t2_ref):
    def corners(base_ref, width_ref, scale_ref):
        w = width_ref[...]
        denom = jnp.maximum(jnp.sum(jnp.abs(w), axis=-1, keepdims=True), 1e-12)
        s = scale_ref[...]
        elu1 = jnp.where(s > 0, s, jnp.exp(jnp.minimum(s, 0.0)) - 1.0) + 1.0
        delta = jnp.abs((w / denom) * elu1)
        b = base_ref[...]
        return b + delta, b - delta

    hu, hl = corners(rhb, rhw, rhs)
    tu, tl = corners(rtb, rtw, rts)
    box_ref[...] = jnp.concatenate([hu, hl, tu, tl], axis=-1)
    bases = eb[...]
    bumps = ebp[...]
    t1_ref[...] = jnp.concatenate([bases, bumps], axis=-1)
    t2_ref[...] = jnp.concatenate([bumps, bases], axis=-1)


def _make_tables(rhb, rhw, rhs, rtb, rtw, rts, eb, ebp):
    return pl.pallas_call(
        _tables_body,
        out_shape=(
            jax.ShapeDtypeStruct((_NB_REL, 4 * _EMB), jnp.float32),
            jax.ShapeDtypeStruct((_NB_ENT, 2 * _EMB), jnp.float32),
            jax.ShapeDtypeStruct((_NB_ENT, 2 * _EMB), jnp.float32),
        ),
    )(rhb, rhw, rhs, rtb, rtw, rts, eb, ebp)


def _sc_body(pos, neg, box_t, t1, t2,
             pos_ent, pos_box, neg_ent, neg_box,
             hbuf, rbuf, tbuf, e1, e2, bbuf, sem):
    wid = lax.axis_index("s") * _NC + lax.axis_index("c")

    def do_chunk(src, n, col0, ent_out, box_out, row0):
        pltpu.sync_copy(src.at[n, 0, pl.ds(col0, _C)], hbuf)
        pltpu.sync_copy(src.at[n, 1, pl.ds(col0, _C)], rbuf)
        pltpu.sync_copy(src.at[n, 2, pl.ds(col0, _C)], tbuf)
        pltpu.async_copy(box_t.at[rbuf], bbuf, sem).wait()
        pltpu.sync_copy(bbuf, box_out.at[pl.ds(row0, _C)])
        pltpu.async_copy(t1.at[hbuf], e1, sem).wait()
        pltpu.async_copy(t2.at[tbuf], e2, sem).wait()

        def add_row(i, carry):
            for k in range(2 * _EMB // 16):
                plsc.addupdate(e1.at[i, pl.ds(16 * k, 16)],
                               e2[i, pl.ds(16 * k, 16)])
            return carry

        lax.fori_loop(0, _C, add_row, 0)
        pltpu.sync_copy(e1, ent_out.at[pl.ds(row0, _C)])

    for cc in range(_BATCH // _NW // _C):
        base = wid * (_BATCH // _NW) + cc * _C
        do_chunk(pos, 0, base, pos_ent, pos_box, base)

    def neg_chunk(j, n):
        do_chunk(neg, n, j * _C, neg_ent, neg_box, n * _BATCH + j * _C)
        return n

    for rr in range(_NB_NEG // _NW):
        n = wid * (_NB_NEG // _NW) + rr
        lax.fori_loop(0, _BATCH // _C, neg_chunk, n)


@functools.cache
def _sc_run():
  return functools.partial(
    pl.kernel,
    mesh=plsc.VectorSubcoreMesh(core_axis_name="c", subcore_axis_name="s"),
    out_type=[
        jax.ShapeDtypeStruct((_BATCH, 2 * _EMB), jnp.float32),
        jax.ShapeDtypeStruct((_BATCH, 4 * _EMB), jnp.float32),
        jax.ShapeDtypeStruct((_NB_NEG * _BATCH, 2 * _EMB), jnp.float32),
        jax.ShapeDtypeStruct((_NB_NEG * _BATCH, 4 * _EMB), jnp.float32),
    ],
    scratch_types=[
        pltpu.VMEM((_C,), jnp.int32),
        pltpu.VMEM((_C,), jnp.int32),
        pltpu.VMEM((_C,), jnp.int32),
        pltpu.VMEM((_C, 2 * _EMB), jnp.float32),
        pltpu.VMEM((_C, 2 * _EMB), jnp.float32),
        pltpu.VMEM((_C, 4 * _EMB), jnp.float32),
        pltpu.SemaphoreType.DMA,
    ],
  )(_sc_body)


def kernel(positives, negatives, r_head_base_points, r_head_widths,
           r_head_size_scales, r_tail_base_points, r_tail_widths,
           r_tail_size_scales, entity_bases, entity_bumps):
    box_t, t1, t2 = _make_tables(
        r_head_base_points, r_head_widths, r_head_size_scales,
        r_tail_base_points, r_tail_widths, r_tail_size_scales,
        entity_bases, entity_bumps)
    pos_ent, pos_box, neg_ent, neg_box = _sc_run()(
        positives, negatives, box_t, t1, t2)
    return (
        pos_ent.reshape(1, _BATCH, 2, _EMB),
        pos_box.reshape(1, _BATCH, 2, 2, _EMB),
        neg_ent.reshape(_NB_NEG, _BATCH, 2, _EMB),
        neg_box.reshape(_NB_NEG, _BATCH, 2, 2, _EMB),
    )


# speedup vs baseline: 6.8620x; 1.0987x over previous
import functools

import jax
import jax.numpy as jnp
from jax import lax
from jax.experimental import pallas as pl
from jax.experimental.pallas import tpu as pltpu
from jax.experimental.pallas import tpu_sc as plsc

_EMB = 128
_NB_REL = 100
_NB_ENT = 1000
_BATCH = 4096
_NB_NEG = 64

_NC = 2
_NS = 16
_NW = _NC * _NS
_C = 32
_POS_PER_W = _BATCH // _NW
_NEG_ROWS_PER_W = _NB_NEG // _NW
_NEG_CHUNKS = _NEG_ROWS_PER_W * _BATCH // _C


def _tables_body(rhb, rhw, rhs, rtb, rtw, rts, eb, ebp,
                 box_ref, t1_ref, t2_ref):
    def corners(base_ref, width_ref, scale_ref):
        w = width_ref[...]
        denom = jnp.maximum(jnp.sum(jnp.abs(w), axis=-1, keepdims=True), 1e-12)
        s = scale_ref[...]
        elu1 = jnp.where(s > 0, s, jnp.exp(jnp.minimum(s, 0.0)) - 1.0) + 1.0
        delta = jnp.abs((w / denom) * elu1)
        b = base_ref[...]
        return b + delta, b - delta

    hu, hl = corners(rhb, rhw, rhs)
    tu, tl = corners(rtb, rtw, rts)
    box_ref[...] = jnp.concatenate([hu, hl, tu, tl], axis=-1)
    bases = eb[...]
    bumps = ebp[...]
    t1_ref[...] = jnp.concatenate([bases, bumps], axis=-1)
    t2_ref[...] = jnp.concatenate([bumps, bases], axis=-1)


def _make_tables(rhb, rhw, rhs, rtb, rtw, rts, eb, ebp):
    return pl.pallas_call(
        _tables_body,
        out_shape=(
            jax.ShapeDtypeStruct((_NB_REL, 4 * _EMB), jnp.float32),
            jax.ShapeDtypeStruct((_NB_ENT, 2 * _EMB), jnp.float32),
            jax.ShapeDtypeStruct((_NB_ENT, 2 * _EMB), jnp.float32),
        ),
    )(rhb, rhw, rhs, rtb, rtw, rts, eb, ebp)


def _sc_body(pos, neg, box_t, t1, t2,
             pos_ent, pos_box, neg_ent, neg_box,
             pih, pir, pit, nih, nir, nit,
             bb0, bb1, ea0, ea1, eb0, eb1,
             bsem0, bsem1, esem0, esem1, wsem0, wsem1):
    bb = (bb0, bb1)
    ea = (ea0, ea1)
    ebuf = (eb0, eb1)
    bsem = (bsem0, bsem1)
    esem = (esem0, esem1)
    wsem = (wsem0, wsem1)

    wid = lax.axis_index("s") * _NC + lax.axis_index("c")
    n0 = wid * _NEG_ROWS_PER_W

    pltpu.sync_copy(pos.at[pl.ds(0 * _BATCH + wid * _POS_PER_W, _POS_PER_W)], pih)
    pltpu.sync_copy(pos.at[pl.ds(1 * _BATCH + wid * _POS_PER_W, _POS_PER_W)], pir)
    pltpu.sync_copy(pos.at[pl.ds(2 * _BATCH + wid * _POS_PER_W, _POS_PER_W)], pit)
    for rr in range(_NEG_ROWS_PER_W):
        dst = pl.ds(rr * _BATCH, _BATCH)
        src0 = (n0 + rr) * 3 * _BATCH
        pltpu.sync_copy(neg.at[pl.ds(src0 + 0 * _BATCH, _BATCH)], nih.at[dst])
        pltpu.sync_copy(neg.at[pl.ds(src0 + 1 * _BATCH, _BATCH)], nir.at[dst])
        pltpu.sync_copy(neg.at[pl.ds(src0 + 2 * _BATCH, _BATCH)], nit.at[dst])

    def issue(slot, ih, ir, it, off, first):
        if not first:
            pltpu.make_async_copy(bb[slot], neg_box.at[pl.ds(0, _C)],
                                  wsem[slot]).wait()
            pltpu.make_async_copy(ea[slot], neg_ent.at[pl.ds(0, _C)],
                                  wsem[slot]).wait()
        pltpu.async_copy(box_t.at[ir.at[pl.ds(off, _C)]], bb[slot], bsem[slot])
        pltpu.async_copy(t1.at[ih.at[pl.ds(off, _C)]], ea[slot], esem[slot])
        pltpu.async_copy(t2.at[it.at[pl.ds(off, _C)]], ebuf[slot], esem[slot])

    def finish(slot, ent_out, box_out, row0):
        pltpu.make_async_copy(box_t.at[pl.ds(0, _C)], bb[slot],
                              bsem[slot]).wait()
        pltpu.async_copy(bb[slot], box_out.at[pl.ds(row0, _C)], wsem[slot])
        pltpu.make_async_copy(t1.at[pl.ds(0, _C)], ea[slot], esem[slot]).wait()
        pltpu.make_async_copy(t2.at[pl.ds(0, _C)], ebuf[slot],
                              esem[slot]).wait()

        def add_row(i, carry):
            for k in range(2 * _EMB // 16):
                plsc.addupdate(ea[slot].at[i, pl.ds(16 * k, 16)],
                               ebuf[slot][i, pl.ds(16 * k, 16)])
            return carry

        lax.fori_loop(0, _C, add_row, 0)
        pltpu.async_copy(ea[slot], ent_out.at[pl.ds(row0, _C)], wsem[slot])

    pbase = wid * _POS_PER_W
    issue(0, pih, pir, pit, 0, True)
    issue(1, pih, pir, pit, _C, True)
    finish(0, pos_ent, pos_box, pbase)
    issue(0, pih, pir, pit, 2 * _C, False)
    finish(1, pos_ent, pos_box, pbase + _C)
    issue(1, pih, pir, pit, 3 * _C, False)
    finish(0, pos_ent, pos_box, pbase + 2 * _C)
    issue(0, nih, nir, nit, 0, False)
    finish(1, pos_ent, pos_box, pbase + 3 * _C)
    issue(1, nih, nir, nit, _C, False)

    nrow0 = n0 * _BATCH

    def step(gg, carry):
        g = gg * 2
        finish(0, neg_ent, neg_box, nrow0 + g * _C)
        issue(0, nih, nir, nit, (g + 2) * _C, False)
        finish(1, neg_ent, neg_box, nrow0 + (g + 1) * _C)
        issue(1, nih, nir, nit, (g + 3) * _C, False)
        return carry

    lax.fori_loop(0, (_NEG_CHUNKS - 2) // 2, step, 0)
    finish(0, neg_ent, neg_box, nrow0 + (_NEG_CHUNKS - 2) * _C)
    finish(1, neg_ent, neg_box, nrow0 + (_NEG_CHUNKS - 1) * _C)
    for slot in (0, 1):
        pltpu.make_async_copy(bb[slot], neg_box.at[pl.ds(0, _C)],
                              wsem[slot]).wait()
        pltpu.make_async_copy(ea[slot], neg_ent.at[pl.ds(0, _C)],
                              wsem[slot]).wait()


@functools.cache
def _sc_run():
  return functools.partial(
    pl.kernel,
    mesh=plsc.VectorSubcoreMesh(core_axis_name="c", subcore_axis_name="s"),
    out_type=[
        jax.ShapeDtypeStruct((_BATCH, 2 * _EMB), jnp.float32),
        jax.ShapeDtypeStruct((_BATCH, 4 * _EMB), jnp.float32),
        jax.ShapeDtypeStruct((_NB_NEG * _BATCH, 2 * _EMB), jnp.float32),
        jax.ShapeDtypeStruct((_NB_NEG * _BATCH, 4 * _EMB), jnp.float32),
    ],
    scratch_types=[
        pltpu.VMEM((_POS_PER_W,), jnp.int32),
        pltpu.VMEM((_POS_PER_W,), jnp.int32),
        pltpu.VMEM((_POS_PER_W,), jnp.int32),
        pltpu.VMEM((_NEG_ROWS_PER_W * _BATCH,), jnp.int32),
        pltpu.VMEM((_NEG_ROWS_PER_W * _BATCH,), jnp.int32),
        pltpu.VMEM((_NEG_ROWS_PER_W * _BATCH,), jnp.int32),
        pltpu.VMEM((_C, 4 * _EMB), jnp.float32),
        pltpu.VMEM((_C, 4 * _EMB), jnp.float32),
        pltpu.VMEM((_C, 2 * _EMB), jnp.float32),
        pltpu.VMEM((_C, 2 * _EMB), jnp.float32),
        pltpu.VMEM((_C, 2 * _EMB), jnp.float32),
        pltpu.VMEM((_C, 2 * _EMB), jnp.float32),
        pltpu.SemaphoreType.DMA,
        pltpu.SemaphoreType.DMA,
        pltpu.SemaphoreType.DMA,
        pltpu.SemaphoreType.DMA,
        pltpu.SemaphoreType.DMA,
        pltpu.SemaphoreType.DMA,
    ],
  )(_sc_body)


def kernel(positives, negatives, r_head_base_points, r_head_widths,
           r_head_size_scales, r_tail_base_points, r_tail_widths,
           r_tail_size_scales, entity_bases, entity_bumps):
    box_t, t1, t2 = _make_tables(
        r_head_base_points, r_head_widths, r_head_size_scales,
        r_tail_base_points, r_tail_widths, r_tail_size_scales,
        entity_bases, entity_bumps)
    pos_ent, pos_box, neg_ent, neg_box = _sc_run()(
        positives.reshape(-1), negatives.reshape(-1), box_t, t1, t2)
    return (
        pos_ent.reshape(1, _BATCH, 2, _EMB),
        pos_box.reshape(1, _BATCH, 2, 2, _EMB),
        neg_ent.reshape(_NB_NEG, _BATCH, 2, _EMB),
        neg_box.reshape(_NB_NEG, _BATCH, 2, 2, _EMB),
    )
